# Initial kernel scaffold; baseline (speedup 1.0000x reference)
import functools

import jax
import jax.numpy as jnp
from jax import lax
from jax.experimental import pallas as pl
from jax.experimental.pallas import tpu as pltpu
from jax.experimental.pallas import tpu_sc as plsc

N_NODES = 10000
N_EDGES = 320000
D = 128

NUM_CORES = 2
NUM_SUBCORES = 16
NUM_WORKERS = NUM_CORES * NUM_SUBCORES
EDGES_PER_WORKER = N_EDGES // NUM_WORKERS
CHUNK = 80
CHUNKS_PER_WORKER = EDGES_PER_WORKER // CHUNK
ROWS_PER_TILE = N_NODES // NUM_SUBCORES

ROW_BLOCK = 1000


def _matmul_body(x_ref, w_ref, b_ref, h_ref):
    h_ref[...] = (
        lax.dot_general(
            x_ref[...], w_ref[...],
            (((1,), (1,)), ((), ())),
            preferred_element_type=jnp.float32,
        )
        + b_ref[...]
    )


def _linear(x, W, b):
    grid = N_NODES // ROW_BLOCK
    return pl.pallas_call(
        _matmul_body,
        grid=(grid,),
        in_specs=[
            pl.BlockSpec((ROW_BLOCK, D), lambda i: (i, 0)),
            pl.BlockSpec((D, D), lambda i: (0, 0)),
            pl.BlockSpec((1, D), lambda i: (0, 0)),
        ],
        out_specs=pl.BlockSpec((ROW_BLOCK, D), lambda i: (i, 0)),
        out_shape=jax.ShapeDtypeStruct((N_NODES, D), jnp.float32),
    )(x, W, b.reshape(1, D))


def _combine_body(a_ref, b_ref, o_ref):
    o_ref[...] = a_ref[0] + b_ref[0]


def _combine(partials):
    grid = N_NODES // ROW_BLOCK
    return pl.pallas_call(
        _combine_body,
        grid=(grid,),
        in_specs=[
            pl.BlockSpec((1, ROW_BLOCK, D), lambda i: (0, i, 0)),
            pl.BlockSpec((1, ROW_BLOCK, D), lambda i: (1, i, 0)),
        ],
        out_specs=pl.BlockSpec((ROW_BLOCK, D), lambda i: (i, 0)),
        out_shape=jax.ShapeDtypeStruct((N_NODES, D), jnp.float32),
    )(partials, partials)


def _scatter_body(h_hbm, src_hbm, dst_hbm, out_hbm,
                  idx_src, idx_dst, rows, zbuf, acc, gsem):
    cid = lax.axis_index("c")
    sid = lax.axis_index("s")
    wid = sid * NUM_CORES + cid
    edge_base = wid * EDGES_PER_WORKER
    row_base = sid * ROWS_PER_TILE

    @pl.when(cid == 0)
    def _():
        pltpu.sync_copy(h_hbm.at[pl.ds(row_base, ROWS_PER_TILE)],
                        acc.at[pl.ds(row_base, ROWS_PER_TILE)])

    @pl.when(cid == 1)
    def _():
        def zvec(i, _):
            r = i // (D // 16)
            c = (i % (D // 16)) * 16
            zbuf[r, pl.ds(c, 16)] = jnp.zeros((16,), jnp.float32)
            return 0
        lax.fori_loop(0, (ROWS_PER_TILE // 5) * (D // 16), zvec, 0)
        for p in range(5):
            pltpu.sync_copy(
                zbuf,
                acc.at[pl.ds(row_base + p * (ROWS_PER_TILE // 5),
                             ROWS_PER_TILE // 5)])

    plsc.subcore_barrier()

    def chunk(j, _):
        off = edge_base + j * CHUNK
        pltpu.sync_copy(src_hbm.at[pl.ds(off, CHUNK)], idx_src)
        pltpu.async_copy(h_hbm.at[idx_src], rows, gsem).wait()
        pltpu.sync_copy(dst_hbm.at[pl.ds(off, CHUNK)], idx_dst)
        pltpu.sync_copy(rows, acc.at[idx_dst], add=True)
        return 0

    lax.fori_loop(0, CHUNKS_PER_WORKER, chunk, 0)

    plsc.subcore_barrier()

    pltpu.sync_copy(acc.at[pl.ds(row_base, ROWS_PER_TILE)],
                    out_hbm.at[cid, pl.ds(row_base, ROWS_PER_TILE)])


@functools.partial(
    pl.kernel,
    out_type=jax.ShapeDtypeStruct((NUM_CORES, N_NODES, D), jnp.float32),
    mesh=plsc.VectorSubcoreMesh(
        core_axis_name="c", subcore_axis_name="s",
        num_cores=NUM_CORES, num_subcores=NUM_SUBCORES),
    scratch_types=[
        pltpu.VMEM((CHUNK,), jnp.int32),
        pltpu.VMEM((CHUNK,), jnp.int32),
        pltpu.VMEM((CHUNK, D), jnp.float32),
        pltpu.VMEM((ROWS_PER_TILE // 5, D), jnp.float32),
        pltpu.VMEM_SHARED((N_NODES, D), jnp.float32),
        pltpu.SemaphoreType.DMA,
    ],
)
def _scatter_add(h_hbm, src_hbm, dst_hbm, out_hbm,
                 idx_src, idx_dst, rows, zbuf, acc, gsem):
    _scatter_body(h_hbm, src_hbm, dst_hbm, out_hbm,
                  idx_src, idx_dst, rows, zbuf, acc, gsem)


def kernel(x, edge_index, W, b):
    src = edge_index[0].astype(jnp.int32)
    dst = edge_index[1].astype(jnp.int32)
    h = _linear(x, W, b)
    partials = _scatter_add(h, src, dst)
    return _combine(partials)


# speedup vs baseline: 7.5457x; 7.5457x over previous
import functools

import jax
import jax.numpy as jnp
from jax import lax
from jax.experimental import pallas as pl
from jax.experimental.pallas import tpu as pltpu
from jax.experimental.pallas import tpu_sc as plsc

N_NODES = 10000
N_EDGES = 320000
D = 128

NUM_CORES = 2
NUM_SUBCORES = 16
NUM_WORKERS = NUM_CORES * NUM_SUBCORES
EDGES_PER_WORKER = N_EDGES // NUM_WORKERS
CHUNK = 80
CHUNKS_PER_WORKER = EDGES_PER_WORKER // CHUNK
ROWS_PER_TILE = 624
TAIL_ROWS = N_NODES - NUM_SUBCORES * ROWS_PER_TILE
ZCHUNK = ROWS_PER_TILE // 4

ROW_BLOCK = 1000


def _matmul_body(x_ref, w_ref, b_ref, h_ref):
    h_ref[...] = (
        lax.dot_general(
            x_ref[...], w_ref[...],
            (((1,), (1,)), ((), ())),
            preferred_element_type=jnp.float32,
        )
        + b_ref[...]
    )


def _linear(x, W, b):
    grid = N_NODES // ROW_BLOCK
    return pl.pallas_call(
        _matmul_body,
        grid=(grid,),
        in_specs=[
            pl.BlockSpec((ROW_BLOCK, D), lambda i: (i, 0)),
            pl.BlockSpec((D, D), lambda i: (0, 0)),
            pl.BlockSpec((1, D), lambda i: (0, 0)),
        ],
        out_specs=pl.BlockSpec((ROW_BLOCK, D), lambda i: (i, 0)),
        out_shape=jax.ShapeDtypeStruct((N_NODES, D), jnp.float32),
    )(x, W, b.reshape(1, D))


def _combine_body(a_ref, b_ref, o_ref):
    o_ref[...] = a_ref[0] + b_ref[0]


def _combine(partials):
    grid = N_NODES // ROW_BLOCK
    return pl.pallas_call(
        _combine_body,
        grid=(grid,),
        in_specs=[
            pl.BlockSpec((1, ROW_BLOCK, D), lambda i: (0, i, 0)),
            pl.BlockSpec((1, ROW_BLOCK, D), lambda i: (1, i, 0)),
        ],
        out_specs=pl.BlockSpec((ROW_BLOCK, D), lambda i: (i, 0)),
        out_shape=jax.ShapeDtypeStruct((N_NODES, D), jnp.float32),
    )(partials, partials)


def _scatter_body(h_hbm, src_hbm, dst_hbm, out_hbm,
                  idx_src, idx_dst, rows, zbuf, acc, gsem):
    cid = lax.axis_index("c")
    sid = lax.axis_index("s")
    wid = sid * NUM_CORES + cid
    edge_base = wid * EDGES_PER_WORKER
    row_base = sid * ROWS_PER_TILE

    @pl.when(cid == 0)
    def _():
        pltpu.sync_copy(h_hbm.at[pl.ds(row_base, ROWS_PER_TILE)],
                        acc.at[pl.ds(row_base, ROWS_PER_TILE)])

        @pl.when(sid == NUM_SUBCORES - 1)
        def _():
            pltpu.sync_copy(
                h_hbm.at[pl.ds(NUM_SUBCORES * ROWS_PER_TILE, TAIL_ROWS)],
                acc.at[pl.ds(NUM_SUBCORES * ROWS_PER_TILE, TAIL_ROWS)])

    @pl.when(cid == 1)
    def _():
        def zvec(i, _):
            r = i // (D // 16)
            c = (i % (D // 16)) * 16
            zbuf[r, pl.ds(c, 16)] = jnp.zeros((16,), jnp.float32)
            return 0
        lax.fori_loop(0, ZCHUNK * (D // 16), zvec, 0)
        for p in range(4):
            pltpu.sync_copy(
                zbuf, acc.at[pl.ds(row_base + p * ZCHUNK, ZCHUNK)])

        @pl.when(sid == NUM_SUBCORES - 1)
        def _():
            pltpu.sync_copy(
                zbuf.at[pl.ds(0, TAIL_ROWS)],
                acc.at[pl.ds(NUM_SUBCORES * ROWS_PER_TILE, TAIL_ROWS)])

    plsc.subcore_barrier()

    def chunk(j, _):
        off = edge_base + j * CHUNK
        pltpu.sync_copy(src_hbm.at[pl.ds(off, CHUNK)], idx_src)
        pltpu.async_copy(h_hbm.at[idx_src], rows, gsem).wait()
        pltpu.sync_copy(dst_hbm.at[pl.ds(off, CHUNK)], idx_dst)
        pltpu.sync_copy(rows, acc.at[idx_dst], add=True)
        return 0

    lax.fori_loop(0, CHUNKS_PER_WORKER, chunk, 0)

    plsc.subcore_barrier()

    pltpu.sync_copy(acc.at[pl.ds(row_base, ROWS_PER_TILE)],
                    out_hbm.at[cid, pl.ds(row_base, ROWS_PER_TILE)])

    @pl.when(sid == NUM_SUBCORES - 1)
    def _():
        pltpu.sync_copy(
            acc.at[pl.ds(NUM_SUBCORES * ROWS_PER_TILE, TAIL_ROWS)],
            out_hbm.at[cid, pl.ds(NUM_SUBCORES * ROWS_PER_TILE, TAIL_ROWS)])


@functools.partial(
    pl.kernel,
    out_type=jax.ShapeDtypeStruct((NUM_CORES, N_NODES, D), jnp.float32),
    mesh=plsc.VectorSubcoreMesh(
        core_axis_name="c", subcore_axis_name="s",
        num_cores=NUM_CORES, num_subcores=NUM_SUBCORES),
    scratch_types=[
        pltpu.VMEM((CHUNK,), jnp.int32),
        pltpu.VMEM((CHUNK,), jnp.int32),
        pltpu.VMEM((CHUNK, D), jnp.float32),
        pltpu.VMEM((ZCHUNK, D), jnp.float32),
        pltpu.VMEM_SHARED((N_NODES, D), jnp.float32),
        pltpu.SemaphoreType.DMA,
    ],
)
def _scatter_add(h_hbm, src_hbm, dst_hbm, out_hbm,
                 idx_src, idx_dst, rows, zbuf, acc, gsem):
    _scatter_body(h_hbm, src_hbm, dst_hbm, out_hbm,
                  idx_src, idx_dst, rows, zbuf, acc, gsem)


def kernel(x, edge_index, W, b):
    src = edge_index[0].astype(jnp.int32)
    dst = edge_index[1].astype(jnp.int32)
    h = _linear(x, W, b)
    partials = _scatter_add(h, src, dst)
    return _combine(partials)


# speedup vs baseline: 14.8447x; 1.9673x over previous
import functools

import jax
import jax.numpy as jnp
from jax import lax
from jax.experimental import pallas as pl
from jax.experimental.pallas import tpu as pltpu
from jax.experimental.pallas import tpu_sc as plsc

N_NODES = 10000
N_EDGES = 320000
D = 128

NUM_CORES = 2
NUM_SUBCORES = 16
NUM_WORKERS = NUM_CORES * NUM_SUBCORES
EDGES_PER_WORKER = N_EDGES // NUM_WORKERS
CHUNK = 40
CHUNKS = EDGES_PER_WORKER // CHUNK
U = 8
OUTER = 248 // U
NROWS = 4
ROWS_PER_TILE = 624
TAIL_ROWS = N_NODES - NUM_SUBCORES * ROWS_PER_TILE

ROW_BLOCK = 1000


def _matmul_body(x_ref, w_ref, b_ref, h_ref):
    h_ref[...] = (
        lax.dot_general(
            x_ref[...], w_ref[...],
            (((1,), (1,)), ((), ())),
            preferred_element_type=jnp.float32,
        )
        + b_ref[...]
    )


def _linear(x, W, b):
    grid = N_NODES // ROW_BLOCK
    return pl.pallas_call(
        _matmul_body,
        grid=(grid,),
        in_specs=[
            pl.BlockSpec((ROW_BLOCK, D), lambda i: (i, 0)),
            pl.BlockSpec((D, D), lambda i: (0, 0)),
            pl.BlockSpec((1, D), lambda i: (0, 0)),
        ],
        out_specs=pl.BlockSpec((ROW_BLOCK, D), lambda i: (i, 0)),
        out_shape=jax.ShapeDtypeStruct((N_NODES, D), jnp.float32),
    )(x, W, b.reshape(1, D))


def _combine_body(a_ref, b_ref, h_ref, o_ref):
    o_ref[...] = a_ref[0] + b_ref[0] - h_ref[...]


def _combine(partials, h):
    grid = N_NODES // ROW_BLOCK
    return pl.pallas_call(
        _combine_body,
        grid=(grid,),
        in_specs=[
            pl.BlockSpec((1, ROW_BLOCK, D), lambda i: (0, i, 0)),
            pl.BlockSpec((1, ROW_BLOCK, D), lambda i: (1, i, 0)),
            pl.BlockSpec((ROW_BLOCK, D), lambda i: (i, 0)),
        ],
        out_specs=pl.BlockSpec((ROW_BLOCK, D), lambda i: (i, 0)),
        out_shape=jax.ShapeDtypeStruct((N_NODES, D), jnp.float32),
    )(partials, partials, h)


def _scatter_body(h_hbm, src_hbm, dst_hbm, out_hbm,
                  sidx, didx, rows, acc, isem, gsem, ssem):
    cid = lax.axis_index("c")
    sid = lax.axis_index("s")
    wid = sid * NUM_CORES + cid
    row_base = sid * ROWS_PER_TILE
    ebase = wid * EDGES_PER_WORKER

    def fire_idx(c, slot):
        pltpu.async_copy(src_hbm.at[pl.ds(ebase + c * CHUNK, CHUNK)],
                         sidx.at[slot], isem)
        pltpu.async_copy(dst_hbm.at[pl.ds(ebase + c * CHUNK, CHUNK)],
                         didx.at[slot], isem)

    def wait_idx():
        pltpu.make_async_copy(
            src_hbm.at[pl.ds(ebase, CHUNK)], sidx.at[0], isem).wait()
        pltpu.make_async_copy(
            src_hbm.at[pl.ds(ebase, CHUNK)], didx.at[0], isem).wait()

    def wait_gather(b):
        pltpu.make_async_copy(h_hbm.at[sidx.at[0]], rows[b], gsem).wait()

    def wait_scatter(b):
        pltpu.make_async_copy(rows[b], acc.at[didx.at[0]], ssem).wait()

    for c in range(NROWS):
        fire_idx(c, c)

    pltpu.sync_copy(h_hbm.at[pl.ds(row_base, ROWS_PER_TILE)],
                    acc.at[pl.ds(row_base, ROWS_PER_TILE)])

    @pl.when(sid == NUM_SUBCORES - 1)
    def _():
        pltpu.sync_copy(
            h_hbm.at[pl.ds(NUM_SUBCORES * ROWS_PER_TILE, TAIL_ROWS)],
            acc.at[pl.ds(NUM_SUBCORES * ROWS_PER_TILE, TAIL_ROWS)])

    wait_idx()
    pltpu.async_copy(h_hbm.at[sidx.at[0]], rows[0], gsem)

    plsc.subcore_barrier()

    def step(j, u):
        @pl.when(j >= 3)
        def _():
            wait_scatter((u + 1) % NROWS)

        @pl.when(j + NROWS < CHUNKS)
        def _():
            fire_idx(j + NROWS, (u + NROWS) % U)

        wait_idx()
        pltpu.async_copy(h_hbm.at[sidx.at[(u + 1) % U]],
                         rows[(u + 1) % NROWS], gsem)
        wait_gather(u % NROWS)
        pltpu.async_copy(rows[u % NROWS], acc.at[didx.at[u]],
                         ssem, add=True)

    def outer(jo, _):
        for u in range(U):
            step(jo * U + u, u)
        return 0

    lax.fori_loop(0, OUTER, outer, 0)

    wait_scatter(1)
    wait_idx()
    pltpu.async_copy(h_hbm.at[sidx.at[1]], rows[1], gsem)
    wait_gather(0)
    pltpu.async_copy(rows[0], acc.at[didx.at[0]], ssem, add=True)
    wait_scatter(2)
    wait_gather(1)
    pltpu.async_copy(rows[1], acc.at[didx.at[1]], ssem, add=True)

    for b in range(3):
        wait_scatter(b)

    plsc.subcore_barrier()

    pltpu.sync_copy(acc.at[pl.ds(row_base, ROWS_PER_TILE)],
                    out_hbm.at[cid, pl.ds(row_base, ROWS_PER_TILE)])

    @pl.when(sid == NUM_SUBCORES - 1)
    def _():
        pltpu.sync_copy(
            acc.at[pl.ds(NUM_SUBCORES * ROWS_PER_TILE, TAIL_ROWS)],
            out_hbm.at[cid, pl.ds(NUM_SUBCORES * ROWS_PER_TILE, TAIL_ROWS)])


@functools.partial(
    pl.kernel,
    out_type=jax.ShapeDtypeStruct((NUM_CORES, N_NODES, D), jnp.float32),
    mesh=plsc.VectorSubcoreMesh(
        core_axis_name="c", subcore_axis_name="s",
        num_cores=NUM_CORES, num_subcores=NUM_SUBCORES),
    scratch_types=[
        pltpu.VMEM((U, CHUNK), jnp.int32),
        pltpu.VMEM((U, CHUNK), jnp.int32),
        [pltpu.VMEM((CHUNK, D), jnp.float32) for _ in range(NROWS)],
        pltpu.VMEM_SHARED((N_NODES, D), jnp.float32),
        pltpu.SemaphoreType.DMA,
        pltpu.SemaphoreType.DMA,
        pltpu.SemaphoreType.DMA,
    ],
)
def _scatter_add(h_hbm, src_hbm, dst_hbm, out_hbm,
                 sidx, didx, rows, acc, isem, gsem, ssem):
    _scatter_body(h_hbm, src_hbm, dst_hbm, out_hbm,
                  sidx, didx, rows, acc, isem, gsem, ssem)


def kernel(x, edge_index, W, b):
    src = edge_index[0].astype(jnp.int32)
    dst = edge_index[1].astype(jnp.int32)
    h = _linear(x, W, b)
    partials = _scatter_add(h, src, dst)
    return _combine(partials, h)


# speedup vs baseline: 18.2245x; 1.2277x over previous
import functools

import jax
import jax.numpy as jnp
from jax import lax
from jax.experimental import pallas as pl
from jax.experimental.pallas import tpu as pltpu
from jax.experimental.pallas import tpu_sc as plsc

N_NODES = 10000
N_EDGES = 320000
D = 128

NUM_CORES = 2
NUM_SUBCORES = 16
NUM_WORKERS = NUM_CORES * NUM_SUBCORES
EDGES_PER_WORKER = N_EDGES // NUM_WORKERS
CHUNK = 80
CHUNKS = EDGES_PER_WORKER // CHUNK
U = 6
OUTER = 120 // U
NROWS = 3
IDXR = 6
ROWS_PER_TILE = 624
TAIL_ROWS = N_NODES - NUM_SUBCORES * ROWS_PER_TILE

ROW_BLOCK = 1000


def _matmul_body(x_ref, w_ref, b_ref, h_ref):
    h_ref[...] = (
        lax.dot_general(
            x_ref[...], w_ref[...],
            (((1,), (1,)), ((), ())),
            preferred_element_type=jnp.float32,
        )
        + b_ref[...]
    )


def _linear(x, W, b):
    grid = N_NODES // ROW_BLOCK
    return pl.pallas_call(
        _matmul_body,
        grid=(grid,),
        in_specs=[
            pl.BlockSpec((ROW_BLOCK, D), lambda i: (i, 0)),
            pl.BlockSpec((D, D), lambda i: (0, 0)),
            pl.BlockSpec((1, D), lambda i: (0, 0)),
        ],
        out_specs=pl.BlockSpec((ROW_BLOCK, D), lambda i: (i, 0)),
        out_shape=jax.ShapeDtypeStruct((N_NODES, D), jnp.float32),
    )(x, W, b.reshape(1, D))


def _combine_body(a_ref, b_ref, h_ref, o_ref):
    o_ref[...] = a_ref[0] + b_ref[0] - h_ref[...]


def _combine(partials, h):
    grid = N_NODES // ROW_BLOCK
    return pl.pallas_call(
        _combine_body,
        grid=(grid,),
        in_specs=[
            pl.BlockSpec((1, ROW_BLOCK, D), lambda i: (0, i, 0)),
            pl.BlockSpec((1, ROW_BLOCK, D), lambda i: (1, i, 0)),
            pl.BlockSpec((ROW_BLOCK, D), lambda i: (i, 0)),
        ],
        out_specs=pl.BlockSpec((ROW_BLOCK, D), lambda i: (i, 0)),
        out_shape=jax.ShapeDtypeStruct((N_NODES, D), jnp.float32),
    )(partials, partials, h)


def _scatter_body(h_hbm, src_hbm, dst_hbm, out_hbm,
                  sidx, didx, rows, acc, isem, gsem, ssem):
    cid = lax.axis_index("c")
    sid = lax.axis_index("s")
    wid = sid * NUM_CORES + cid
    row_base = sid * ROWS_PER_TILE
    ebase = wid * EDGES_PER_WORKER

    def fire_idx(c, slot):
        pltpu.async_copy(src_hbm.at[pl.ds(ebase + c * CHUNK, CHUNK)],
                         sidx.at[slot], isem)
        pltpu.async_copy(dst_hbm.at[pl.ds(ebase + c * CHUNK, CHUNK)],
                         didx.at[slot], isem)

    def wait_idx():
        pltpu.make_async_copy(
            src_hbm.at[pl.ds(ebase, CHUNK)], sidx.at[0], isem).wait()
        pltpu.make_async_copy(
            src_hbm.at[pl.ds(ebase, CHUNK)], didx.at[0], isem).wait()

    def wait_gather(b):
        pltpu.make_async_copy(h_hbm.at[sidx.at[0]], rows[b], gsem).wait()

    def wait_scatter(b):
        pltpu.make_async_copy(rows[b], acc.at[didx.at[0]], ssem).wait()

    for c in range(4):
        fire_idx(c, c)

    pltpu.sync_copy(h_hbm.at[pl.ds(row_base, ROWS_PER_TILE)],
                    acc.at[pl.ds(row_base, ROWS_PER_TILE)])

    @pl.when(sid == NUM_SUBCORES - 1)
    def _():
        pltpu.sync_copy(
            h_hbm.at[pl.ds(NUM_SUBCORES * ROWS_PER_TILE, TAIL_ROWS)],
            acc.at[pl.ds(NUM_SUBCORES * ROWS_PER_TILE, TAIL_ROWS)])

    wait_idx()
    pltpu.async_copy(h_hbm.at[sidx.at[0]], rows[0], gsem)

    plsc.subcore_barrier()

    def step(j, u, guard_lo, fire_i, fire_g):
        if guard_lo:
            @pl.when(j >= 2)
            def _():
                wait_scatter((u + 1) % NROWS)
        else:
            wait_scatter((u + 1) % NROWS)

        if fire_i == "when":
            @pl.when(j + 4 < CHUNKS)
            def _():
                fire_idx(j + 4, (u + 4) % IDXR)
        elif fire_i:
            fire_idx(j + 4, (u + 4) % IDXR)

        if fire_g:
            wait_idx()
            pltpu.async_copy(h_hbm.at[sidx.at[(u + 1) % IDXR]],
                             rows[(u + 1) % NROWS], gsem)
        wait_gather(u % NROWS)
        pltpu.async_copy(rows[u % NROWS], acc.at[didx.at[u % IDXR]],
                         ssem, add=True)

    def outer(jo, _):
        for u in range(U):
            step(jo * U + u, u, guard_lo=True, fire_i=True, fire_g=True)
        return 0

    lax.fori_loop(0, OUTER, outer, 0)

    step(120, 0, guard_lo=False, fire_i=True, fire_g=True)
    step(121, 1, guard_lo=False, fire_i=False, fire_g=True)
    step(122, 2, guard_lo=False, fire_i=False, fire_g=True)
    step(123, 3, guard_lo=False, fire_i=False, fire_g=True)
    step(124, 4, guard_lo=False, fire_i=False, fire_g=False)

    for b in range(2):
        wait_scatter(b)

    plsc.subcore_barrier()

    pltpu.sync_copy(acc.at[pl.ds(row_base, ROWS_PER_TILE)],
                    out_hbm.at[cid, pl.ds(row_base, ROWS_PER_TILE)])

    @pl.when(sid == NUM_SUBCORES - 1)
    def _():
        pltpu.sync_copy(
            acc.at[pl.ds(NUM_SUBCORES * ROWS_PER_TILE, TAIL_ROWS)],
            out_hbm.at[cid, pl.ds(NUM_SUBCORES * ROWS_PER_TILE, TAIL_ROWS)])


@functools.partial(
    pl.kernel,
    out_type=jax.ShapeDtypeStruct((NUM_CORES, N_NODES, D), jnp.float32),
    mesh=plsc.VectorSubcoreMesh(
        core_axis_name="c", subcore_axis_name="s",
        num_cores=NUM_CORES, num_subcores=NUM_SUBCORES),
    scratch_types=[
        pltpu.VMEM((U, CHUNK), jnp.int32),
        pltpu.VMEM((U, CHUNK), jnp.int32),
        [pltpu.VMEM((CHUNK, D), jnp.float32) for _ in range(NROWS)],
        pltpu.VMEM_SHARED((N_NODES, D), jnp.float32),
        pltpu.SemaphoreType.DMA,
        pltpu.SemaphoreType.DMA,
        pltpu.SemaphoreType.DMA,
    ],
)
def _scatter_add(h_hbm, src_hbm, dst_hbm, out_hbm,
                 sidx, didx, rows, acc, isem, gsem, ssem):
    _scatter_body(h_hbm, src_hbm, dst_hbm, out_hbm,
                  sidx, didx, rows, acc, isem, gsem, ssem)


def kernel(x, edge_index, W, b):
    src = edge_index[0].astype(jnp.int32)
    dst = edge_index[1].astype(jnp.int32)
    h = _linear(x, W, b)
    partials = _scatter_add(h, src, dst)
    return _combine(partials, h)


# speedup vs baseline: 21.4507x; 1.1770x over previous
import functools

import jax
import jax.numpy as jnp
from jax import lax
from jax.experimental import pallas as pl
from jax.experimental.pallas import tpu as pltpu
from jax.experimental.pallas import tpu_sc as plsc

N_NODES = 10000
N_EDGES = 320000
D = 128

NUM_CORES = 2
NUM_SUBCORES = 16
NUM_WORKERS = NUM_CORES * NUM_SUBCORES
CHUNK = 128
CHUNKS = 78
LEFT_BASE = NUM_WORKERS * CHUNKS
U = 6
OUTER = 72 // U
NROWS = 3
IDXR = 6
ROWS_PER_TILE = 624
TAIL_ROWS = N_NODES - NUM_SUBCORES * ROWS_PER_TILE

ROW_BLOCK = 2000


def _matmul_body(x_ref, w_ref, b_ref, h_ref):
    h_ref[...] = (
        lax.dot_general(
            x_ref[...], w_ref[...],
            (((1,), (1,)), ((), ())),
            preferred_element_type=jnp.float32,
        )
        + b_ref[...]
    )


def _linear(x, W, b):
    grid = N_NODES // ROW_BLOCK
    return pl.pallas_call(
        _matmul_body,
        grid=(grid,),
        in_specs=[
            pl.BlockSpec((ROW_BLOCK, D), lambda i: (i, 0)),
            pl.BlockSpec((D, D), lambda i: (0, 0)),
            pl.BlockSpec((1, D), lambda i: (0, 0)),
        ],
        out_specs=pl.BlockSpec((ROW_BLOCK, D), lambda i: (i, 0)),
        out_shape=jax.ShapeDtypeStruct((N_NODES, D), jnp.float32),
    )(x, W, b.reshape(1, D))


def _combine_body(a_ref, b_ref, h_ref, o_ref):
    o_ref[...] = a_ref[0] + b_ref[0] - h_ref[...]


def _combine(partials, h):
    grid = N_NODES // ROW_BLOCK
    return pl.pallas_call(
        _combine_body,
        grid=(grid,),
        in_specs=[
            pl.BlockSpec((1, ROW_BLOCK, D), lambda i: (0, i, 0)),
            pl.BlockSpec((1, ROW_BLOCK, D), lambda i: (1, i, 0)),
            pl.BlockSpec((ROW_BLOCK, D), lambda i: (i, 0)),
        ],
        out_specs=pl.BlockSpec((ROW_BLOCK, D), lambda i: (i, 0)),
        out_shape=jax.ShapeDtypeStruct((N_NODES, D), jnp.float32),
    )(partials, partials, h)


def _scatter_body(h_hbm, ei_hbm, out_hbm, idx, rows, acc, isem, gsem, ssem):
    cid = lax.axis_index("c")
    sid = lax.axis_index("s")
    wid = sid * NUM_CORES + cid
    row_base = sid * ROWS_PER_TILE
    cbase = wid * CHUNKS

    def fire_idx(c, slot):
        pltpu.async_copy(
            ei_hbm.at[:, pl.ds((cbase + c) * CHUNK, CHUNK)],
            idx.at[slot], isem)

    def wait_idx():
        pltpu.make_async_copy(
            ei_hbm.at[:, pl.ds(0, CHUNK)], idx.at[0], isem).wait()

    def wait_gather(b):
        pltpu.make_async_copy(h_hbm.at[idx.at[0, 0]], rows[b], gsem).wait()

    def wait_scatter(b):
        pltpu.make_async_copy(rows[b], acc.at[idx.at[0, 1]], ssem).wait()

    for c in range(4):
        fire_idx(c, c)

    pltpu.sync_copy(h_hbm.at[pl.ds(row_base, ROWS_PER_TILE)],
                    acc.at[pl.ds(row_base, ROWS_PER_TILE)])

    @pl.when(sid == NUM_SUBCORES - 1)
    def _():
        pltpu.sync_copy(
            h_hbm.at[pl.ds(NUM_SUBCORES * ROWS_PER_TILE, TAIL_ROWS)],
            acc.at[pl.ds(NUM_SUBCORES * ROWS_PER_TILE, TAIL_ROWS)])

    wait_idx()
    pltpu.async_copy(h_hbm.at[idx.at[0, 0]], rows[0], gsem)

    plsc.subcore_barrier()

    def step(j, u, guard_lo, fire_i, fire_g):
        if guard_lo:
            @pl.when(j >= 2)
            def _():
                wait_scatter((u + 1) % NROWS)
        else:
            wait_scatter((u + 1) % NROWS)

        if fire_i:
            fire_idx(j + 4, (u + 4) % IDXR)

        if fire_g:
            wait_idx()
            pltpu.async_copy(h_hbm.at[idx.at[(u + 1) % IDXR, 0]],
                             rows[(u + 1) % NROWS], gsem)
        wait_gather(u % NROWS)
        pltpu.async_copy(rows[u % NROWS], acc.at[idx.at[u % IDXR, 1]],
                         ssem, add=True)

    def outer(jo, _):
        for u in range(U):
            step(jo * U + u, u, guard_lo=True, fire_i=True, fire_g=True)
        return 0

    lax.fori_loop(0, OUTER, outer, 0)

    step(72, 0, guard_lo=False, fire_i=True, fire_g=True)
    step(73, 1, guard_lo=False, fire_i=True, fire_g=True)
    step(74, 2, guard_lo=False, fire_i=False, fire_g=True)
    step(75, 3, guard_lo=False, fire_i=False, fire_g=True)
    step(76, 4, guard_lo=False, fire_i=False, fire_g=True)
    step(77, 5, guard_lo=False, fire_i=False, fire_g=False)

    for b in range(2):
        wait_scatter(b)

    @pl.when(wid < 2500 - LEFT_BASE)
    def _():
        pltpu.sync_copy(
            ei_hbm.at[:, pl.ds((LEFT_BASE + wid) * CHUNK, CHUNK)],
            idx.at[0])
        pltpu.async_copy(h_hbm.at[idx.at[0, 0]], rows[0], gsem).wait()
        pltpu.async_copy(rows[0], acc.at[idx.at[0, 1]], ssem, add=True)
        wait_scatter(0)

    plsc.subcore_barrier()

    pltpu.sync_copy(acc.at[pl.ds(row_base, ROWS_PER_TILE)],
                    out_hbm.at[cid, pl.ds(row_base, ROWS_PER_TILE)])

    @pl.when(sid == NUM_SUBCORES - 1)
    def _():
        pltpu.sync_copy(
            acc.at[pl.ds(NUM_SUBCORES * ROWS_PER_TILE, TAIL_ROWS)],
            out_hbm.at[cid, pl.ds(NUM_SUBCORES * ROWS_PER_TILE, TAIL_ROWS)])


@functools.partial(
    pl.kernel,
    out_type=jax.ShapeDtypeStruct((NUM_CORES, N_NODES, D), jnp.float32),
    mesh=plsc.VectorSubcoreMesh(
        core_axis_name="c", subcore_axis_name="s",
        num_cores=NUM_CORES, num_subcores=NUM_SUBCORES),
    scratch_types=[
        pltpu.VMEM((IDXR, 2, CHUNK), jnp.int32),
        [pltpu.VMEM((CHUNK, D), jnp.float32) for _ in range(NROWS)],
        pltpu.VMEM_SHARED((N_NODES, D), jnp.float32),
        pltpu.SemaphoreType.DMA,
        pltpu.SemaphoreType.DMA,
        pltpu.SemaphoreType.DMA,
    ],
)
def _scatter_add(h_hbm, ei_hbm, out_hbm, idx, rows, acc, isem, gsem, ssem):
    _scatter_body(h_hbm, ei_hbm, out_hbm, idx, rows, acc, isem, gsem, ssem)


def kernel(x, edge_index, W, b):
    h = _linear(x, W, b)
    partials = _scatter_add(h, edge_index.astype(jnp.int32))
    return _combine(partials, h)
